# baseline (speedup 1.0000x reference)
import functools
import math

import jax
import jax.numpy as jnp
from jax import lax
from jax.experimental import pallas as pl
from jax.experimental.pallas import tpu as pltpu
from jax.experimental.pallas import tpu_sc as plsc

D = 768
H = 12
HD = 64
W = 384
LMAX = 2048
KMAX = 6
PW = 400
DP = 384
GSLOT = 384
SCALE = 1.0 / math.sqrt(HD)
NEG = -1e30


def _proj_body(xq_ref, xk_ref, xv_ref, wq_ref, wk_ref, wv_ref,
               q_ref, k_ref, v_ref, r_ref):
    dn = (((1,), (1,)), ((), ()))
    q = lax.dot_general(xq_ref[...], wq_ref[...], dn,
                        preferred_element_type=jnp.float32)
    k = lax.dot_general(xk_ref[...], wk_ref[...], dn,
                        preferred_element_type=jnp.float32)
    v = lax.dot_general(xv_ref[...], wv_ref[...], dn,
                        preferred_element_type=jnp.float32)
    q_ref[...] = q.astype(jnp.bfloat16)
    k_ref[...] = k.astype(jnp.bfloat16)
    v_ref[...] = v.astype(jnp.bfloat16)

    def headmean(x):
        acc = x[:, 0:HD]
        for h in range(1, H):
            acc = acc + x[:, h * HD:(h + 1) * HD]
        return acc * (1.0 / H)

    def l2n(x):
        n = jnp.sqrt(jnp.sum(x * x, axis=-1, keepdims=True))
        return x / (n + 1e-6)

    r_ref[...] = 0.5 * (l2n(headmean(q)) + l2n(headmean(k)))


def _project(q_in, k_in, v_in, Wq, Wk, Wv):
    T = q_in.shape[0]
    BR = 1024
    grid = (T // BR,)
    row = pl.BlockSpec((BR, D), lambda i: (i, 0))
    wsp = pl.BlockSpec((D, D), lambda i: (0, 0))
    return pl.pallas_call(
        _proj_body,
        grid=grid,
        in_specs=[row, row, row, wsp, wsp, wsp],
        out_specs=[row, row, row, pl.BlockSpec((BR, HD), lambda i: (i, 0))],
        out_shape=[
            jax.ShapeDtypeStruct((T, D), jnp.bfloat16),
            jax.ShapeDtypeStruct((T, D), jnp.bfloat16),
            jax.ShapeDtypeStruct((T, D), jnp.bfloat16),
            jax.ShapeDtypeStruct((T, HD), jnp.float32),
        ],
    )(q_in, k_in, v_in, Wq, Wk, Wv)


def _route_plan(r_pad, lens, gidx, Ttot):
    S = lens.shape[0]
    lens = lens.astype(jnp.int32)
    cu = jnp.concatenate([jnp.zeros((1,), jnp.int32), jnp.cumsum(lens)])
    a = jnp.minimum(cu[:-1], Ttot)
    b = jnp.minimum(cu[1:], Ttot)
    Ls = b - a
    w = jnp.maximum(1, jnp.minimum(W, Ls))
    k = jnp.maximum(1, (Ls + w - 1) // w)
    jl = jnp.arange(LMAX, dtype=jnp.int32)
    valid = jl[None, :] < Ls[:, None]
    feats = jnp.take(r_pad, a[:, None] + jl[None, :], axis=0)
    fv = jnp.where(valid[:, :, None], feats, 0.0)
    jc = jnp.arange(KMAX, dtype=jnp.int32)
    cmask = jc[None, :] < k[:, None]
    Lm1 = jnp.maximum(Ls - 1, 0)
    denom = jnp.maximum(k - 1, 1).astype(jnp.float32)
    pts = jc[None, :].astype(jnp.float32) * (Lm1.astype(jnp.float32) / denom)[:, None]
    init_idx = jnp.clip(jnp.round(pts).astype(jnp.int32), 0, Lm1[:, None])
    centroids = jnp.take_along_axis(feats, init_idx[:, :, None], axis=1)
    for _ in range(1):
        sims = jnp.einsum('sld,skd->slk', feats, centroids)
        sims = jnp.where(cmask[:, None, :], sims, -jnp.inf)
        assign = jnp.argmax(sims, axis=-1)
        oh = ((assign[:, :, None] == jc[None, None, :]) & valid[:, :, None]
              ).astype(jnp.float32)
        sums = jnp.einsum('slk,sld->skd', oh, fv)
        counts = jnp.maximum(oh.sum(axis=1), 1.0)
        centroids = sums / counts[:, :, None]
        centroids = centroids / (
            jnp.linalg.norm(centroids, axis=-1, keepdims=True) + 1e-6)
    sims = jnp.einsum('sld,skd->slk', feats, centroids)
    sims = jnp.where(cmask[:, None, :], sims, -jnp.inf)
    assign = jnp.argmax(sims, axis=-1)
    own = jnp.take_along_axis(sims, assign[:, :, None], axis=-1)[:, :, 0]
    key1 = jnp.where(valid, -own, jnp.inf)
    order1 = jnp.argsort(key1, axis=-1, stable=True)
    assign1 = jnp.take_along_axis(assign, order1, axis=-1)
    valid1 = jnp.take_along_axis(valid, order1, axis=-1)
    key2 = jnp.where(valid1, assign1, KMAX)
    order2 = jnp.argsort(key2, axis=-1, stable=True)
    perm = jnp.take_along_axis(order1, order2, axis=-1)
    cid_sorted = jnp.take_along_axis(key2, order2, axis=-1)
    oh2 = ((cid_sorted[:, :, None] == jc[None, None, :]) & valid[:, :, None]
           ).astype(jnp.int32)
    counts_c = oh2.sum(axis=1)
    starts = jnp.cumsum(counts_c, axis=1) - counts_c
    cid_cl = jnp.clip(cid_sorted, 0, KMAX - 1).astype(jnp.int32)
    pos = jl[None, :] - jnp.take_along_axis(starts, cid_cl, axis=1)
    keep = valid & (pos < w[:, None])
    abs_tok = (a[:, None] + perm).astype(jnp.int32)
    s_grid = jnp.broadcast_to(jnp.arange(S, dtype=jnp.int32)[:, None], (S, LMAX))
    p_cl = jnp.clip(pos, 0, W - 1).astype(jnp.int32)

    idx3 = jnp.zeros((S, KMAX, PW), jnp.int32).at[
        s_grid, cid_cl, p_cl].add(jnp.where(keep, abs_tok, 0))
    msk3 = jnp.zeros((S, KMAX, PW), jnp.int32).at[
        s_grid, cid_cl, p_cl].add(keep.astype(jnp.int32))
    cluster_ok = counts_c > 0
    gi = gidx.astype(jnp.int32)
    idx3 = idx3.at[:, :, GSLOT].set(jnp.broadcast_to(gi[:, None], (S, KMAX)))
    msk3 = msk3.at[:, :, GSLOT].set(cluster_ok.astype(jnp.int32))

    nslot = S * KMAX * PW
    slot = (s_grid * KMAX + cid_cl) * PW + p_cl
    scat = jnp.where(keep, abs_tok, Ttot)
    inv_slot = jnp.full((Ttot,), nslot, jnp.int32).at[scat].set(
        slot.astype(jnp.int32), mode='drop')
    kept = jnp.zeros((Ttot,), jnp.float32).at[scat].set(1.0, mode='drop')
    g_cnt = cluster_ok.astype(jnp.float32).sum(axis=1)
    return (idx3.reshape(S * KMAX * PW), msk3.reshape(S * KMAX, 1, PW),
            inv_slot, kept, g_cnt)


def _sc_gather(tables, idx, chunk, width):
    n_tab = len(tables)
    B = idx.shape[0]
    NW = 32
    bpw = B // NW
    nch = bpw // chunk
    mesh = plsc.VectorSubcoreMesh(core_axis_name="c", subcore_axis_name="s")

    bufs_t = [pltpu.VMEM((chunk, width), jnp.int32) for _ in range(2 * n_tab)]
    sems_t = [pltpu.SemaphoreType.DMA for _ in range(2 * n_tab)]

    @functools.partial(
        pl.kernel, mesh=mesh,
        out_type=[jax.ShapeDtypeStruct((B, width), jnp.int32)] * n_tab,
        scratch_types=[pltpu.VMEM((nch, chunk), jnp.int32)] + bufs_t + sems_t,
    )
    def run(*refs):
        tabs = refs[:n_tab]
        idx_hbm = refs[n_tab]
        outs = refs[n_tab + 1:2 * n_tab + 1]
        scr = refs[2 * n_tab + 1:]
        idx_v = scr[0]
        bufs = scr[1:1 + 2 * n_tab]
        sems = scr[1 + 2 * n_tab:]
        wid = lax.axis_index("s") * 2 + lax.axis_index("c")
        wbase = pl.multiple_of(wid * bpw, 8)
        pltpu.sync_copy(idx_hbm.at[wid], idx_v)

        def chunk_refs(p, j, i):
            off = pl.multiple_of(i * chunk, 8)
            buf = bufs[p * n_tab + j]
            sem = sems[p * n_tab + j]
            src = tabs[j].at[idx_v.at[i]]
            dst = outs[j].at[pl.ds(wbase + off, chunk)]
            return buf, sem, src, dst

        def start_gather(p, j, i):
            buf, sem, src, _ = chunk_refs(p, j, i)
            pltpu.async_copy(src, buf, sem)

        def start_store(p, j, i):
            buf, sem, _, dst = chunk_refs(p, j, i)
            pltpu.async_copy(buf, dst, sem)

        def wait_on(p, j, i):
            buf, sem, src, _ = chunk_refs(p, j, i)
            pltpu.make_async_copy(src, buf, sem).wait()

        def body(i, carry):
            p = lax.rem(i, 2)
            for j in range(n_tab):
                for ph in range(2):
                    @pl.when((p == ph) & (i < nch) & (i >= 2))
                    def _():
                        wait_on(ph, j, i - 2)
                    @pl.when((p == ph) & (i < nch))
                    def _():
                        start_gather(ph, j, i)
            for j in range(n_tab):
                for ph in range(2):
                    @pl.when((1 - p == ph) & (i >= 1))
                    def _():
                        wait_on(ph, j, i - 1)
                        start_store(ph, j, i - 1)
            return carry

        lax.fori_loop(0, nch + 1, body, 0)
        for j in range(n_tab):
            for last in (nch - 1, nch - 2):
                if last >= 0:
                    wait_on(last % 2, j, last)

    NW = 32
    return run(*tables, idx.reshape(NW, nch, chunk))


def _attn_body(msk_ref, qg_ref, kg_ref, vg_ref, out_ref):
    mk = msk_ref[0]
    q = qg_ref[0]
    k = kg_ref[0]
    v = vg_ref[0]
    bias = (mk - 1.0) * -NEG
    outs = []
    for h in range(H):
        sl = slice(h * HD, (h + 1) * HD)
        s = lax.dot_general(q[:, sl], k[:, sl],
                            (((1,), (1,)), ((), ())),
                            preferred_element_type=jnp.float32)
        s = s * SCALE + bias
        m = jnp.max(s, axis=-1, keepdims=True)
        p = jnp.exp(s - m)
        den = jnp.sum(p, axis=-1, keepdims=True)
        o = lax.dot_general(p.astype(jnp.bfloat16), v[:, sl],
                            (((1,), (0,)), ((), ())),
                            preferred_element_type=jnp.float32)
        outs.append(o / den)
    res = jnp.concatenate(outs, axis=1)
    res = res * jnp.transpose(mk)
    out_ref[0] = res.astype(jnp.bfloat16)


def _attention(msk, qg, kg, vg):
    C = qg.shape[0]
    blk = pl.BlockSpec((1, PW, D), lambda i: (i, 0, 0))
    return pl.pallas_call(
        _attn_body,
        grid=(C,),
        in_specs=[pl.BlockSpec((1, 1, PW), lambda i: (i, 0, 0)),
                  blk, blk, blk],
        out_specs=blk,
        out_shape=jax.ShapeDtypeStruct((C, PW, D), jnp.bfloat16),
    )(msk, qg, kg, vg)


def _final_body(og_ref, kept_ref, gsum_ref, gcnt_ref, gidx_ref, wo_ref,
                out_ref, *, br):
    i = pl.program_id(0)
    rows = lax.broadcasted_iota(jnp.int32, (br, 16), 0) + i * br
    onehot = (rows == gidx_ref[0:1, :]).astype(jnp.float32)
    gadd = lax.dot_general(onehot, gsum_ref[...], (((1,), (0,)), ((), ())),
                           preferred_element_type=jnp.float32)
    cadd = jnp.sum(onehot * gcnt_ref[0:1, :], axis=-1, keepdims=True)
    kept = jnp.max(kept_ref[...], axis=-1, keepdims=True)
    cnt = jnp.maximum(kept + cadd, 1.0)
    merged = (og_ref[...].astype(jnp.float32) + gadd) / cnt
    out_ref[...] = lax.dot_general(merged, wo_ref[...], (((1,), (1,)), ((), ())),
                                   preferred_element_type=jnp.float32)


def _finalize(out_g, kept8, g_sum, g_cnt8, gidx8, Wo):
    T = out_g.shape[0]
    BR = 1024
    return pl.pallas_call(
        functools.partial(_final_body, br=BR),
        grid=(T // BR,),
        in_specs=[pl.BlockSpec((BR, D), lambda i: (i, 0)),
                  pl.BlockSpec((BR, 8), lambda i: (i, 0)),
                  pl.BlockSpec((16, D), lambda i: (0, 0)),
                  pl.BlockSpec((8, 16), lambda i: (0, 0)),
                  pl.BlockSpec((8, 16), lambda i: (0, 0)),
                  pl.BlockSpec((D, D), lambda i: (0, 0))],
        out_specs=pl.BlockSpec((BR, D), lambda i: (i, 0)),
        out_shape=jax.ShapeDtypeStruct((T, D), jnp.float32),
    )(out_g, kept8, g_sum, g_cnt8, gidx8, Wo)


def kernel(q_in, k_in, v_in, seqlens_tokens, global_idx_per_set, Wq, Wk, Wv, Wo):
    Ttot = q_in.shape[0]
    S = seqlens_tokens.shape[0]

    q_full, k_full, v_full, r = _project(q_in, k_in, v_in, Wq, Wk, Wv)
    r_pad = jnp.concatenate([r, jnp.zeros((LMAX, HD), jnp.float32)], axis=0)

    idx_flat, msk, inv_slot, kept, g_cnt = _route_plan(
        r_pad, seqlens_tokens, global_idx_per_set, Ttot)

    def to_i32(x):
        return lax.bitcast_convert_type(
            x.reshape(x.shape[0], DP, 2), jnp.int32)

    def to_bf16(x):
        return lax.bitcast_convert_type(x, jnp.bfloat16).reshape(x.shape[0], D)

    qg, kg, vg = _sc_gather(
        [to_i32(q_full), to_i32(k_full), to_i32(v_full)],
        idx_flat, chunk=40, width=DP)
    nslot = S * KMAX * PW
    packed = _attention(msk.astype(jnp.float32),
                        to_bf16(qg).reshape(S * KMAX, PW, D),
                        to_bf16(kg).reshape(S * KMAX, PW, D),
                        to_bf16(vg).reshape(S * KMAX, PW, D))

    packed_i32 = to_i32(packed.reshape(nslot, D))
    packed_pad = jnp.concatenate(
        [packed_i32, jnp.zeros((8, DP), jnp.int32)], axis=0)
    (out_gi,) = _sc_gather([packed_pad], inv_slot, chunk=64, width=DP)
    out_g = to_bf16(out_gi)

    g_rows = packed[:, GSLOT, :]
    g_sum = g_rows.reshape(S, KMAX, D).astype(jnp.float32).sum(axis=1)
    kept8 = jnp.broadcast_to(kept[:, None], (Ttot, 8))
    g_cnt8 = jnp.broadcast_to(g_cnt[None, :], (8, S))
    gidx8 = jnp.broadcast_to(global_idx_per_set.astype(jnp.int32)[None, :], (8, S))

    return _finalize(out_g, kept8, g_sum, g_cnt8, gidx8, Wo)


# speedup vs baseline: 1.6980x; 1.6980x over previous
import functools
import math

import jax
import jax.numpy as jnp
from jax import lax
from jax.experimental import pallas as pl
from jax.experimental.pallas import tpu as pltpu
from jax.experimental.pallas import tpu_sc as plsc

D = 768
H = 12
HD = 64
W = 384
LMAX = 2048
KMAX = 6
PW = 400
DP = 384
GSLOT = 384
SCALE = 1.0 / math.sqrt(HD)
NEG = -1e30


def _proj_body(xq_ref, xk_ref, xv_ref, wq_ref, wk_ref, wv_ref,
               qkv_ref, r_ref):
    dn = (((1,), (1,)), ((), ()))
    q = lax.dot_general(xq_ref[...], wq_ref[...], dn,
                        preferred_element_type=jnp.float32)
    k = lax.dot_general(xk_ref[...], wk_ref[...], dn,
                        preferred_element_type=jnp.float32)
    v = lax.dot_general(xv_ref[...], wv_ref[...], dn,
                        preferred_element_type=jnp.float32)
    qkv_ref[:, 0:D] = q
    qkv_ref[:, D:2 * D] = k
    qkv_ref[:, 2 * D:3 * D] = v

    def headmean(x):
        acc = x[:, 0:HD]
        for h in range(1, H):
            acc = acc + x[:, h * HD:(h + 1) * HD]
        return acc * (1.0 / H)

    def l2n(x):
        n = jnp.sqrt(jnp.sum(x * x, axis=-1, keepdims=True))
        return x / (n + 1e-6)

    r_ref[...] = 0.5 * (l2n(headmean(q)) + l2n(headmean(k)))


def _project(q_in, k_in, v_in, Wq, Wk, Wv):
    T = q_in.shape[0]
    BR = 1024
    grid = (T // BR,)
    row = pl.BlockSpec((BR, D), lambda i: (i, 0))
    wsp = pl.BlockSpec((D, D), lambda i: (0, 0))
    return pl.pallas_call(
        _proj_body,
        grid=grid,
        in_specs=[row, row, row, wsp, wsp, wsp],
        out_specs=[pl.BlockSpec((BR, 3 * D), lambda i: (i, 0)),
                   pl.BlockSpec((BR, HD), lambda i: (i, 0))],
        out_shape=[
            jax.ShapeDtypeStruct((T, 3 * D), jnp.float32),
            jax.ShapeDtypeStruct((T, HD), jnp.float32),
        ],
    )(q_in, k_in, v_in, Wq, Wk, Wv)


def _route_plan(r_pad, lens, gidx, Ttot):
    S = lens.shape[0]
    lens = lens.astype(jnp.int32)
    cu = jnp.concatenate([jnp.zeros((1,), jnp.int32), jnp.cumsum(lens)])
    a = jnp.minimum(cu[:-1], Ttot)
    b = jnp.minimum(cu[1:], Ttot)
    Ls = b - a
    w = jnp.maximum(1, jnp.minimum(W, Ls))
    k = jnp.maximum(1, (Ls + w - 1) // w)
    jl = jnp.arange(LMAX, dtype=jnp.int32)
    valid = jl[None, :] < Ls[:, None]
    feats = jnp.take(r_pad, a[:, None] + jl[None, :], axis=0)
    fv = jnp.where(valid[:, :, None], feats, 0.0)
    jc = jnp.arange(KMAX, dtype=jnp.int32)
    cmask = jc[None, :] < k[:, None]
    Lm1 = jnp.maximum(Ls - 1, 0)
    denom = jnp.maximum(k - 1, 1).astype(jnp.float32)
    pts = jc[None, :].astype(jnp.float32) * (Lm1.astype(jnp.float32) / denom)[:, None]
    init_idx = jnp.clip(jnp.round(pts).astype(jnp.int32), 0, Lm1[:, None])
    centroids = jnp.take_along_axis(feats, init_idx[:, :, None], axis=1)
    for _ in range(1):
        sims = jnp.einsum('sld,skd->slk', feats, centroids)
        sims = jnp.where(cmask[:, None, :], sims, -jnp.inf)
        assign = jnp.argmax(sims, axis=-1)
        oh = ((assign[:, :, None] == jc[None, None, :]) & valid[:, :, None]
              ).astype(jnp.float32)
        sums = jnp.einsum('slk,sld->skd', oh, fv)
        counts = jnp.maximum(oh.sum(axis=1), 1.0)
        centroids = sums / counts[:, :, None]
        centroids = centroids / (
            jnp.linalg.norm(centroids, axis=-1, keepdims=True) + 1e-6)
    sims = jnp.einsum('sld,skd->slk', feats, centroids)
    sims = jnp.where(cmask[:, None, :], sims, -jnp.inf)
    assign = jnp.argmax(sims, axis=-1)
    own = jnp.take_along_axis(sims, assign[:, :, None], axis=-1)[:, :, 0]
    key1 = jnp.where(valid, -own, jnp.inf)
    order1 = jnp.argsort(key1, axis=-1, stable=True)
    assign1 = jnp.take_along_axis(assign, order1, axis=-1)
    valid1 = jnp.take_along_axis(valid, order1, axis=-1)
    key2 = jnp.where(valid1, assign1, KMAX)
    order2 = jnp.argsort(key2, axis=-1, stable=True)
    perm = jnp.take_along_axis(order1, order2, axis=-1)
    cid_sorted = jnp.take_along_axis(key2, order2, axis=-1)
    oh2 = ((cid_sorted[:, :, None] == jc[None, None, :]) & valid[:, :, None]
           ).astype(jnp.int32)
    counts_c = oh2.sum(axis=1)
    starts = jnp.cumsum(counts_c, axis=1) - counts_c
    cid_cl = jnp.clip(cid_sorted, 0, KMAX - 1).astype(jnp.int32)
    pos = jl[None, :] - jnp.take_along_axis(starts, cid_cl, axis=1)
    keep = valid & (pos < w[:, None])
    abs_tok = (a[:, None] + perm).astype(jnp.int32)
    s_grid = jnp.broadcast_to(jnp.arange(S, dtype=jnp.int32)[:, None], (S, LMAX))
    p_cl = jnp.clip(pos, 0, W - 1).astype(jnp.int32)

    idx3 = jnp.zeros((S, KMAX, PW), jnp.int32).at[
        s_grid, cid_cl, p_cl].add(jnp.where(keep, abs_tok, 0))
    msk3 = jnp.zeros((S, KMAX, PW), jnp.int32).at[
        s_grid, cid_cl, p_cl].add(keep.astype(jnp.int32))
    cluster_ok = counts_c > 0
    gi = gidx.astype(jnp.int32)
    idx3 = idx3.at[:, :, GSLOT].set(jnp.broadcast_to(gi[:, None], (S, KMAX)))
    msk3 = msk3.at[:, :, GSLOT].set(cluster_ok.astype(jnp.int32))

    nslot = S * KMAX * PW
    slot = (s_grid * KMAX + cid_cl) * PW + p_cl
    scat = jnp.where(keep, abs_tok, Ttot)
    inv_slot = jnp.full((Ttot,), nslot, jnp.int32).at[scat].set(
        slot.astype(jnp.int32), mode='drop')
    kept = jnp.zeros((Ttot,), jnp.float32).at[scat].set(1.0, mode='drop')
    g_cnt = cluster_ok.astype(jnp.float32).sum(axis=1)
    return (idx3.reshape(S * KMAX * PW), msk3.reshape(S * KMAX, 1, PW),
            inv_slot, kept, g_cnt)


def _sc_gather(tables, idx, chunk, width):
    n_tab = len(tables)
    B = idx.shape[0]
    NW = 32
    bpw = B // NW
    nch = bpw // chunk
    mesh = plsc.VectorSubcoreMesh(core_axis_name="c", subcore_axis_name="s")

    bufs_t = [pltpu.VMEM((chunk, width), jnp.float32) for _ in range(2 * n_tab)]
    sems_t = [pltpu.SemaphoreType.DMA for _ in range(2 * n_tab)]

    @functools.partial(
        pl.kernel, mesh=mesh,
        out_type=[jax.ShapeDtypeStruct((B, width), jnp.float32)] * n_tab,
        scratch_types=[pltpu.VMEM((nch, chunk), jnp.int32)] + bufs_t + sems_t,
    )
    def run(*refs):
        tabs = refs[:n_tab]
        idx_hbm = refs[n_tab]
        outs = refs[n_tab + 1:2 * n_tab + 1]
        scr = refs[2 * n_tab + 1:]
        idx_v = scr[0]
        bufs = scr[1:1 + 2 * n_tab]
        sems = scr[1 + 2 * n_tab:]
        wid = lax.axis_index("s") * 2 + lax.axis_index("c")
        wbase = pl.multiple_of(wid * bpw, 8)
        pltpu.sync_copy(idx_hbm.at[wid], idx_v)

        def chunk_refs(p, j, i):
            off = pl.multiple_of(i * chunk, 8)
            buf = bufs[p * n_tab + j]
            sem = sems[p * n_tab + j]
            src = tabs[j].at[idx_v.at[i]]
            dst = outs[j].at[pl.ds(wbase + off, chunk)]
            return buf, sem, src, dst

        def start_gather(p, j, i):
            buf, sem, src, _ = chunk_refs(p, j, i)
            pltpu.async_copy(src, buf, sem)

        def start_store(p, j, i):
            buf, sem, _, dst = chunk_refs(p, j, i)
            pltpu.async_copy(buf, dst, sem)

        def wait_on(p, j, i):
            buf, sem, src, _ = chunk_refs(p, j, i)
            pltpu.make_async_copy(src, buf, sem).wait()

        def body(i, carry):
            p = lax.rem(i, 2)
            for j in range(n_tab):
                for ph in range(2):
                    @pl.when((p == ph) & (i < nch) & (i >= 2))
                    def _():
                        wait_on(ph, j, i - 2)
                    @pl.when((p == ph) & (i < nch))
                    def _():
                        start_gather(ph, j, i)
            for j in range(n_tab):
                for ph in range(2):
                    @pl.when((1 - p == ph) & (i >= 1))
                    def _():
                        wait_on(ph, j, i - 1)
                        start_store(ph, j, i - 1)
            return carry

        lax.fori_loop(0, nch + 1, body, 0)
        for j in range(n_tab):
            for last in (nch - 1, nch - 2):
                if last >= 0:
                    wait_on(last % 2, j, last)

    NW = 32
    return run(*tables, idx.reshape(NW, nch, chunk))


def _attn_body(msk_ref, g_ref, out_ref):
    mk = msk_ref[0]
    x = g_ref[0]
    bias = (mk - 1.0) * -NEG
    outs = []
    for h in range(H):
        qs = slice(h * HD, (h + 1) * HD)
        ks = slice(D + h * HD, D + (h + 1) * HD)
        vs = slice(2 * D + h * HD, 2 * D + (h + 1) * HD)
        s = lax.dot_general(x[:, qs] * SCALE, x[:, ks],
                            (((1,), (1,)), ((), ())),
                            preferred_element_type=jnp.float32)
        s = s + bias
        m = jnp.max(s, axis=-1, keepdims=True)
        p = jnp.exp(s - m)
        den = jnp.sum(p, axis=-1, keepdims=True)
        o = lax.dot_general(p, x[:, vs], (((1,), (0,)), ((), ())),
                            preferred_element_type=jnp.float32)
        outs.append(o / den)
    res = jnp.concatenate(outs, axis=1)
    out_ref[0] = res * jnp.transpose(mk)


def _attention(msk, g):
    C = g.shape[0]
    return pl.pallas_call(
        _attn_body,
        grid=(C,),
        in_specs=[pl.BlockSpec((1, 1, PW), lambda i: (i, 0, 0)),
                  pl.BlockSpec((1, PW, 3 * D), lambda i: (i, 0, 0))],
        out_specs=pl.BlockSpec((1, PW, D), lambda i: (i, 0, 0)),
        out_shape=jax.ShapeDtypeStruct((C, PW, D), jnp.float32),
    )(msk, g)


def _final_body(og_ref, kept_ref, gsum_ref, gcnt_ref, gidx_ref, wo_ref,
                out_ref, *, br):
    i = pl.program_id(0)
    rows = lax.broadcasted_iota(jnp.int32, (br, 16), 0) + i * br
    onehot = (rows == gidx_ref[0:1, :]).astype(jnp.float32)
    gadd = lax.dot_general(onehot, gsum_ref[...], (((1,), (0,)), ((), ())),
                           preferred_element_type=jnp.float32)
    cadd = jnp.sum(onehot * gcnt_ref[0:1, :], axis=-1, keepdims=True)
    kept = jnp.max(kept_ref[...], axis=-1, keepdims=True)
    cnt = jnp.maximum(kept + cadd, 1.0)
    merged = (og_ref[...] + gadd) / cnt
    out_ref[...] = lax.dot_general(merged, wo_ref[...], (((1,), (1,)), ((), ())),
                                   preferred_element_type=jnp.float32)


def _finalize(out_g, kept8, g_sum, g_cnt8, gidx8, Wo):
    T = out_g.shape[0]
    BR = 1024
    return pl.pallas_call(
        functools.partial(_final_body, br=BR),
        grid=(T // BR,),
        in_specs=[pl.BlockSpec((BR, D), lambda i: (i, 0)),
                  pl.BlockSpec((BR, 8), lambda i: (i, 0)),
                  pl.BlockSpec((16, D), lambda i: (0, 0)),
                  pl.BlockSpec((8, 16), lambda i: (0, 0)),
                  pl.BlockSpec((8, 16), lambda i: (0, 0)),
                  pl.BlockSpec((D, D), lambda i: (0, 0))],
        out_specs=pl.BlockSpec((BR, D), lambda i: (i, 0)),
        out_shape=jax.ShapeDtypeStruct((T, D), jnp.float32),
    )(out_g, kept8, g_sum, g_cnt8, gidx8, Wo)


def kernel(q_in, k_in, v_in, seqlens_tokens, global_idx_per_set, Wq, Wk, Wv, Wo):
    Ttot = q_in.shape[0]
    S = seqlens_tokens.shape[0]

    qkv, r = _project(q_in, k_in, v_in, Wq, Wk, Wv)
    r_pad = jnp.concatenate([r, jnp.zeros((LMAX, HD), jnp.float32)], axis=0)

    idx_flat, msk, inv_slot, kept, g_cnt = _route_plan(
        r_pad, seqlens_tokens, global_idx_per_set, Ttot)

    (qkvg,) = _sc_gather([qkv], idx_flat, chunk=24, width=3 * D)
    nslot = S * KMAX * PW
    packed = _attention(msk.astype(jnp.float32),
                        qkvg.reshape(S * KMAX, PW, 3 * D))

    packed_flat = packed.reshape(nslot, D)
    packed_pad = jnp.concatenate(
        [packed_flat, jnp.zeros((8, D), jnp.float32)], axis=0)
    (out_g,) = _sc_gather([packed_pad], inv_slot, chunk=64, width=D)

    g_rows = packed[:, GSLOT, :]
    g_sum = g_rows.reshape(S, KMAX, D).sum(axis=1)
    kept8 = jnp.broadcast_to(kept[:, None], (Ttot, 8))
    g_cnt8 = jnp.broadcast_to(g_cnt[None, :], (8, S))
    gidx8 = jnp.broadcast_to(global_idx_per_set.astype(jnp.int32)[None, :], (8, S))

    return _finalize(out_g, kept8, g_sum, g_cnt8, gidx8, Wo)


# speedup vs baseline: 3.0458x; 1.7938x over previous
import functools
import math

import jax
import jax.numpy as jnp
from jax import lax
from jax.experimental import pallas as pl
from jax.experimental.pallas import tpu as pltpu
from jax.experimental.pallas import tpu_sc as plsc

D = 768
H = 12
HD = 64
W = 384
LMAX = 2048
KMAX = 6
PW = 400
NCL = 16 * KMAX
NSLOT = NCL * PW
CH = 16
NPH = 2
NPC = -(-W // CH)
EMAX = NCL * NPC + NCL
NW = 32
NCMAX = -(-EMAX // NW)
EPAD = NCMAX * NW
GSLOT = 384
SCALE = 1.0 / math.sqrt(HD)
NEG = -1e30


def _proj_body(xq_ref, xk_ref, xv_ref, wq_ref, wk_ref, wv_ref,
               qkv_ref, r_ref):
    dn = (((1,), (1,)), ((), ()))
    q = lax.dot_general(xq_ref[...], wq_ref[...], dn,
                        preferred_element_type=jnp.float32)
    k = lax.dot_general(xk_ref[...], wk_ref[...], dn,
                        preferred_element_type=jnp.float32)
    v = lax.dot_general(xv_ref[...], wv_ref[...], dn,
                        preferred_element_type=jnp.float32)
    qkv_ref[:, 0:D] = q
    qkv_ref[:, D:2 * D] = k
    qkv_ref[:, 2 * D:3 * D] = v

    def headmean(x):
        acc = x[:, 0:HD]
        for h in range(1, H):
            acc = acc + x[:, h * HD:(h + 1) * HD]
        return acc * (1.0 / H)

    def l2n(x):
        n = jnp.sqrt(jnp.sum(x * x, axis=-1, keepdims=True))
        return x / (n + 1e-6)

    r_ref[...] = 0.5 * (l2n(headmean(q)) + l2n(headmean(k)))


def _project(q_in, k_in, v_in, Wq, Wk, Wv):
    T = q_in.shape[0]
    BR = 1024
    grid = (T // BR,)
    row = pl.BlockSpec((BR, D), lambda i: (i, 0))
    wsp = pl.BlockSpec((D, D), lambda i: (0, 0))
    return pl.pallas_call(
        _proj_body,
        grid=grid,
        in_specs=[row, row, row, wsp, wsp, wsp],
        out_specs=[pl.BlockSpec((BR, 3 * D), lambda i: (i, 0)),
                   pl.BlockSpec((BR, HD), lambda i: (i, 0))],
        out_shape=[
            jax.ShapeDtypeStruct((T, 3 * D), jnp.float32),
            jax.ShapeDtypeStruct((T, HD), jnp.float32),
        ],
    )(q_in, k_in, v_in, Wq, Wk, Wv)


def _route_plan(r_pad, lens, gidx, Ttot):
    S = lens.shape[0]
    lens = lens.astype(jnp.int32)
    cu = jnp.concatenate([jnp.zeros((1,), jnp.int32), jnp.cumsum(lens)])
    a = jnp.minimum(cu[:-1], Ttot)
    b = jnp.minimum(cu[1:], Ttot)
    Ls = b - a
    w = jnp.maximum(1, jnp.minimum(W, Ls))
    k = jnp.maximum(1, (Ls + w - 1) // w)
    jl = jnp.arange(LMAX, dtype=jnp.int32)
    valid = jl[None, :] < Ls[:, None]
    feats = jnp.take(r_pad, a[:, None] + jl[None, :], axis=0)
    fv = jnp.where(valid[:, :, None], feats, 0.0)
    jc = jnp.arange(KMAX, dtype=jnp.int32)
    cmask = jc[None, :] < k[:, None]
    Lm1 = jnp.maximum(Ls - 1, 0)
    denom = jnp.maximum(k - 1, 1).astype(jnp.float32)
    pts = jc[None, :].astype(jnp.float32) * (Lm1.astype(jnp.float32) / denom)[:, None]
    init_idx = jnp.clip(jnp.round(pts).astype(jnp.int32), 0, Lm1[:, None])
    centroids = jnp.take_along_axis(feats, init_idx[:, :, None], axis=1)
    for _ in range(1):
        sims = jnp.einsum('sld,skd->slk', feats, centroids)
        sims = jnp.where(cmask[:, None, :], sims, -jnp.inf)
        assign = jnp.argmax(sims, axis=-1)
        oh = ((assign[:, :, None] == jc[None, None, :]) & valid[:, :, None]
              ).astype(jnp.float32)
        sums = jnp.einsum('slk,sld->skd', oh, fv)
        counts = jnp.maximum(oh.sum(axis=1), 1.0)
        centroids = sums / counts[:, :, None]
        centroids = centroids / (
            jnp.linalg.norm(centroids, axis=-1, keepdims=True) + 1e-6)
    sims = jnp.einsum('sld,skd->slk', feats, centroids)
    sims = jnp.where(cmask[:, None, :], sims, -jnp.inf)
    assign = jnp.argmax(sims, axis=-1)
    own = jnp.take_along_axis(sims, assign[:, :, None], axis=-1)[:, :, 0]
    key1 = jnp.where(valid, -own, jnp.inf)
    order1 = jnp.argsort(key1, axis=-1, stable=True)
    assign1 = jnp.take_along_axis(assign, order1, axis=-1)
    valid1 = jnp.take_along_axis(valid, order1, axis=-1)
    key2 = jnp.where(valid1, assign1, KMAX)
    order2 = jnp.argsort(key2, axis=-1, stable=True)
    perm = jnp.take_along_axis(order1, order2, axis=-1)
    cid_sorted = jnp.take_along_axis(key2, order2, axis=-1)
    oh2 = ((cid_sorted[:, :, None] == jc[None, None, :]) & valid[:, :, None]
           ).astype(jnp.int32)
    counts_c = oh2.sum(axis=1)
    starts = jnp.cumsum(counts_c, axis=1) - counts_c
    cid_cl = jnp.clip(cid_sorted, 0, KMAX - 1).astype(jnp.int32)
    pos = jl[None, :] - jnp.take_along_axis(starts, cid_cl, axis=1)
    keep = valid & (pos < w[:, None])
    abs_tok = (a[:, None] + perm).astype(jnp.int32)
    s_grid = jnp.broadcast_to(jnp.arange(S, dtype=jnp.int32)[:, None], (S, LMAX))
    p_cl = jnp.clip(pos, 0, W - 1).astype(jnp.int32)

    idx3 = jnp.zeros((S, KMAX, PW), jnp.int32).at[
        s_grid, cid_cl, p_cl].add(jnp.where(keep, abs_tok, 0))
    msk3 = jnp.zeros((S, KMAX, PW), jnp.int32).at[
        s_grid, cid_cl, p_cl].add(keep.astype(jnp.int32))
    cluster_ok = counts_c > 0
    gi = gidx.astype(jnp.int32)
    idx3 = idx3.at[:, :, GSLOT].set(jnp.broadcast_to(gi[:, None], (S, KMAX)))
    msk3 = msk3.at[:, :, GSLOT].set(cluster_ok.astype(jnp.int32))

    slot = (s_grid * KMAX + cid_cl) * PW + p_cl
    scat = jnp.where(keep, abs_tok, Ttot)
    inv_slot = jnp.full((Ttot,), NSLOT, jnp.int32).at[scat].set(
        slot.astype(jnp.int32), mode='drop')
    kept = jnp.zeros((Ttot,), jnp.float32).at[scat].set(1.0, mode='drop')
    g_cnt = cluster_ok.astype(jnp.float32).sum(axis=1)

    idx_flat = idx3.reshape(NSLOT)
    kcnt = jnp.minimum(counts_c, w[:, None]).reshape(NCL)
    nch_pc = (kcnt + CH - 1) // CH
    cum_incl = jnp.cumsum(nch_pc)
    total_v = cum_incl[-1]
    cum_excl = cum_incl - nch_pc
    e = jnp.arange(EPAD, dtype=jnp.int32)
    c_v = jnp.clip(jnp.searchsorted(cum_incl, e, side='right'),
                   0, NCL - 1).astype(jnp.int32)
    j_v = e - cum_excl[c_v]
    start_valid = c_v * PW + CH * j_v
    is_g = (e >= total_v) & (e < total_v + NCL)
    c_g = jnp.clip(e - total_v, 0, NCL - 1)
    goff = min(GSLOT, PW - CH)
    start = jnp.where(e < total_v, start_valid,
                      jnp.where(is_g, c_g * PW + goff, NSLOT))
    rows = start[:, None] + jnp.arange(CH, dtype=jnp.int32)[None, :]
    real = (e < total_v + NCL)[:, None]
    ch_idx_f = jnp.where(real, idx_flat[jnp.clip(rows, 0, NSLOT - 1)], 0)
    e_grid = jnp.arange(NW, dtype=jnp.int32)[:, None] + \
        NW * jnp.arange(NCMAX, dtype=jnp.int32)[None, :]
    ch_idx = ch_idx_f[e_grid]
    ch_dst = rows[e_grid]
    nw_cnt = jnp.maximum(0, (total_v + NCL - jnp.arange(NW) + NW - 1) // NW)
    nwarr = jnp.broadcast_to(nw_cnt[:, None], (NW, 16)).astype(jnp.float32)

    return (ch_idx, ch_dst, nwarr, msk3.reshape(NCL, 1, PW),
            inv_slot, kept, g_cnt)


def _sc_gather(tables, idx, chunk, width):
    n_tab = len(tables)
    B = idx.shape[0]
    NW = 32
    bpw = B // NW
    nch = bpw // chunk
    mesh = plsc.VectorSubcoreMesh(core_axis_name="c", subcore_axis_name="s")

    bufs_t = [pltpu.VMEM((chunk, width), jnp.float32) for _ in range(2 * n_tab)]
    sems_t = [pltpu.SemaphoreType.DMA for _ in range(2 * n_tab)]

    @functools.partial(
        pl.kernel, mesh=mesh,
        out_type=[jax.ShapeDtypeStruct((B, width), jnp.float32)] * n_tab,
        scratch_types=[pltpu.VMEM((nch, chunk), jnp.int32)] + bufs_t + sems_t,
    )
    def run(*refs):
        tabs = refs[:n_tab]
        idx_hbm = refs[n_tab]
        outs = refs[n_tab + 1:2 * n_tab + 1]
        scr = refs[2 * n_tab + 1:]
        idx_v = scr[0]
        bufs = scr[1:1 + 2 * n_tab]
        sems = scr[1 + 2 * n_tab:]
        wid = lax.axis_index("s") * 2 + lax.axis_index("c")
        wbase = pl.multiple_of(wid * bpw, 8)
        pltpu.sync_copy(idx_hbm.at[wid], idx_v)

        def chunk_refs(p, j, i):
            off = pl.multiple_of(i * chunk, 8)
            buf = bufs[p * n_tab + j]
            sem = sems[p * n_tab + j]
            src = tabs[j].at[idx_v.at[i]]
            dst = outs[j].at[pl.ds(wbase + off, chunk)]
            return buf, sem, src, dst

        def start_gather(p, j, i):
            buf, sem, src, _ = chunk_refs(p, j, i)
            pltpu.async_copy(src, buf, sem)

        def start_store(p, j, i):
            buf, sem, _, dst = chunk_refs(p, j, i)
            pltpu.async_copy(buf, dst, sem)

        def wait_on(p, j, i):
            buf, sem, src, _ = chunk_refs(p, j, i)
            pltpu.make_async_copy(src, buf, sem).wait()

        def body(i, carry):
            p = lax.rem(i, 2)
            for j in range(n_tab):
                for ph in range(2):
                    @pl.when((p == ph) & (i < nch) & (i >= 2))
                    def _():
                        wait_on(ph, j, i - 2)
                    @pl.when((p == ph) & (i < nch))
                    def _():
                        start_gather(ph, j, i)
            for j in range(n_tab):
                for ph in range(2):
                    @pl.when((1 - p == ph) & (i >= 1))
                    def _():
                        wait_on(ph, j, i - 1)
                        start_store(ph, j, i - 1)
            return carry

        lax.fori_loop(0, nch + 1, body, 0)
        for j in range(n_tab):
            for last in (nch - 1, nch - 2):
                if last >= 0:
                    wait_on(last % 2, j, last)

    NW = 32
    return run(*tables, idx.reshape(NW, nch, chunk))


def _sc_pack(qkv, ch_idx, ch_dst, nwarr):
    mesh = plsc.VectorSubcoreMesh(core_axis_name="c", subcore_axis_name="s")
    W3 = 3 * D

    @functools.partial(
        pl.kernel, mesh=mesh,
        out_type=jax.ShapeDtypeStruct(((NCL + 1) * PW, W3), jnp.float32),
        scratch_types=[pltpu.VMEM((NCMAX, CH), jnp.int32),
                       pltpu.VMEM((NCMAX, CH), jnp.int32),
                       pltpu.VMEM((16,), jnp.float32)]
        + [pltpu.VMEM((CH, W3), jnp.float32) for _ in range(NPH)]
        + [pltpu.SemaphoreType.DMA for _ in range(NPH)],
    )
    def run(tab, idx_hbm, dst_hbm, nw_hbm, out, idx_v, dst_v, nw_v, *bs):
        bufs = bs[:NPH]
        sems = bs[NPH:]
        wid = lax.axis_index("s") * 2 + lax.axis_index("c")
        pltpu.sync_copy(idx_hbm.at[wid], idx_v)
        pltpu.sync_copy(dst_hbm.at[wid], dst_v)
        pltpu.sync_copy(nw_hbm.at[wid], nw_v)
        nw = jnp.squeeze(lax.slice(nw_v[...], (0,), (1,))).astype(jnp.int32)

        def wait_any(ph):
            pltpu.make_async_copy(
                bufs[ph], out.at[pl.ds(0, CH)], sems[ph]).wait()

        def body(t, carry):
            for ph in range(NPH):
                @pl.when(((t % NPH) == ph) & (t < nw) & (t >= NPH))
                def _():
                    wait_any(ph)
                @pl.when(((t % NPH) == ph) & (t < nw))
                def _():
                    pltpu.async_copy(tab.at[idx_v.at[t]], bufs[ph], sems[ph])
            for ph in range(NPH):
                @pl.when((t >= 1) & (((t - 1) % NPH) == ph) & (t - 1 < nw))
                def _():
                    wait_any(ph)
                    pltpu.async_copy(bufs[ph], out.at[dst_v.at[t - 1]],
                                     sems[ph])
            return carry

        lax.fori_loop(0, NCMAX + 1, body, 0)
        for ph in range(NPH):
            @pl.when(jnp.minimum(nw, NPH) > ph)
            def _():
                wait_any(ph)

    return run(qkv, ch_idx, ch_dst, nwarr)


def _attn_body(msk_ref, g_ref, out_ref):
    mk = msk_ref[0]
    mq = jnp.transpose(mk)
    x = jnp.where(mq > 0, g_ref[0], 0.0)
    bias = (mk - 1.0) * -NEG
    outs = []
    for h in range(H):
        qs = slice(h * HD, (h + 1) * HD)
        ks = slice(D + h * HD, D + (h + 1) * HD)
        vs = slice(2 * D + h * HD, 2 * D + (h + 1) * HD)
        s = lax.dot_general(x[:, qs] * SCALE, x[:, ks],
                            (((1,), (1,)), ((), ())),
                            preferred_element_type=jnp.float32)
        s = s + bias
        m = jnp.max(s, axis=-1, keepdims=True)
        p = jnp.exp(s - m)
        den = jnp.sum(p, axis=-1, keepdims=True)
        o = lax.dot_general(p, x[:, vs], (((1,), (0,)), ((), ())),
                            preferred_element_type=jnp.float32)
        outs.append(o / den)
    res = jnp.concatenate(outs, axis=1)
    out_ref[0] = res * mq


def _attention(msk, g):
    C = NCL
    return pl.pallas_call(
        _attn_body,
        grid=(C,),
        in_specs=[pl.BlockSpec((1, 1, PW), lambda i: (i, 0, 0)),
                  pl.BlockSpec((1, PW, 3 * D), lambda i: (i, 0, 0))],
        out_specs=pl.BlockSpec((1, PW, D), lambda i: (i, 0, 0)),
        out_shape=jax.ShapeDtypeStruct((C, PW, D), jnp.float32),
    )(msk, g)


def _final_body(og_ref, kept_ref, gsum_ref, gcnt_ref, gidx_ref, wo_ref,
                out_ref, *, br):
    i = pl.program_id(0)
    rows = lax.broadcasted_iota(jnp.int32, (br, 16), 0) + i * br
    onehot = (rows == gidx_ref[0:1, :]).astype(jnp.float32)
    gadd = lax.dot_general(onehot, gsum_ref[...], (((1,), (0,)), ((), ())),
                           preferred_element_type=jnp.float32)
    cadd = jnp.sum(onehot * gcnt_ref[0:1, :], axis=-1, keepdims=True)
    kept = jnp.max(kept_ref[...], axis=-1, keepdims=True)
    cnt = jnp.maximum(kept + cadd, 1.0)
    merged = (og_ref[...] + gadd) / cnt
    out_ref[...] = lax.dot_general(merged, wo_ref[...], (((1,), (1,)), ((), ())),
                                   preferred_element_type=jnp.float32)


def _finalize(out_g, kept8, g_sum, g_cnt8, gidx8, Wo):
    T = out_g.shape[0]
    BR = 1024
    return pl.pallas_call(
        functools.partial(_final_body, br=BR),
        grid=(T // BR,),
        in_specs=[pl.BlockSpec((BR, D), lambda i: (i, 0)),
                  pl.BlockSpec((BR, 8), lambda i: (i, 0)),
                  pl.BlockSpec((16, D), lambda i: (0, 0)),
                  pl.BlockSpec((8, 16), lambda i: (0, 0)),
                  pl.BlockSpec((8, 16), lambda i: (0, 0)),
                  pl.BlockSpec((D, D), lambda i: (0, 0))],
        out_specs=pl.BlockSpec((BR, D), lambda i: (i, 0)),
        out_shape=jax.ShapeDtypeStruct((T, D), jnp.float32),
    )(out_g, kept8, g_sum, g_cnt8, gidx8, Wo)


def kernel(q_in, k_in, v_in, seqlens_tokens, global_idx_per_set, Wq, Wk, Wv, Wo):
    Ttot = q_in.shape[0]
    S = seqlens_tokens.shape[0]

    qkv, r = _project(q_in, k_in, v_in, Wq, Wk, Wv)
    r_pad = jnp.concatenate([r, jnp.zeros((LMAX, HD), jnp.float32)], axis=0)

    ch_idx, ch_dst, nwarr, msk, inv_slot, kept, g_cnt = _route_plan(
        r_pad, seqlens_tokens, global_idx_per_set, Ttot)

    qkvp = _sc_pack(qkv, ch_idx, ch_dst, nwarr)
    packed = _attention(msk.astype(jnp.float32),
                        qkvp.reshape(NCL + 1, PW, 3 * D))

    packed_flat = packed.reshape(NSLOT, D)
    packed_pad = jnp.concatenate(
        [packed_flat, jnp.zeros((8, D), jnp.float32)], axis=0)
    (out_g,) = _sc_gather([packed_pad], inv_slot, chunk=64, width=D)

    g_rows = packed[:, GSLOT, :]
    g_sum = g_rows.reshape(S, KMAX, D).sum(axis=1)
    kept8 = jnp.broadcast_to(kept[:, None], (Ttot, 8))
    g_cnt8 = jnp.broadcast_to(g_cnt[None, :], (8, S))
    gidx8 = jnp.broadcast_to(global_idx_per_set.astype(jnp.int32)[None, :], (8, S))

    return _finalize(out_g, kept8, g_sum, g_cnt8, gidx8, Wo)
